# baseline (speedup 1.0000x reference)
import jax
import jax.numpy as jnp
from jax import lax
from jax.experimental import pallas as pl
from jax.experimental.pallas import tpu as pltpu
from jax.experimental.pallas import tpu_sc as plsc

VOCAB = 1000000
EMB_DIM = 64

NC = 2
NS = 16
NW = NC * NS

BATCH = 4096
SEQ = 200
LANE = 128
SUB = 8
SEQ_T = SEQ // SUB
BATCH_T = BATCH // LANE
NBUF = 4
NROUND = SEQ // NBUF


def _body(idx_hbm, table_hbm, out_hbm, idx_v, gbuf, tbuf, sem_g, sem_w):
    c = lax.axis_index("c")
    s = lax.axis_index("s")
    w = s * NC + c

    pltpu.sync_copy(idx_hbm.at[:, w], idx_v)

    lanes = lax.iota(jnp.int32, 16)

    def remap_tile(si, carry):
        for sj in range(SUB):
            for k in range(LANE // 16):
                sl = idx_v.at[si, sj, pl.ds(k * 16, 16)]
                v = sl[...]
                sl[...] = (v & jnp.int32(~15)) | ((v & 7) << 1) | ((v >> 3) & 1)
        return carry

    lax.fori_loop(0, SEQ_T, remap_tile, 0)

    def gather(sq, b):
        si = sq // SUB
        sj = sq % SUB
        return pltpu.make_async_copy(
            table_hbm.at[idx_v.at[si, sj]], gbuf.at[b], sem_g.at[b]
        )

    def writeback(sq, b):
        return pltpu.make_async_copy(tbuf.at[b], out_hbm.at[sq, :, w], sem_w.at[b])

    def transpose(b):
        def tr_d(d, carry):
            dhi = d // SUB
            dlo = d % SUB
            cols = jnp.broadcast_to(d, (16,))
            for k in range(LANE // 16):
                rows = k * 16 + lanes
                vals = plsc.load_gather(gbuf.at[b], [rows, cols])
                tbuf[b, dhi, dlo, pl.ds(k * 16, 16)] = vals
            return carry

        lax.fori_loop(0, EMB_DIM, tr_d, 0)

    for b in range(NBUF):
        gather(b, b).start()

    def ring(i, carry):
        for b in range(NBUF):
            sq = i * NBUF + b
            gather(sq, b).wait()

            @pl.when(i > 0)
            def _():
                writeback(sq, b).wait()

            transpose(b)

            @pl.when(i < NROUND - 1)
            def _():
                gather(sq + NBUF, b).start()

            writeback(sq, b).start()
        return carry

    lax.fori_loop(0, NROUND, ring, 0)

    for b in range(NBUF):
        writeback(0, b).wait()


@jax.jit
def _embed(idx4, table_p):
    mesh = plsc.VectorSubcoreMesh(core_axis_name="c", subcore_axis_name="s")
    run = pl.kernel(
        _body,
        out_type=jax.ShapeDtypeStruct((SEQ, SUB, BATCH_T, SUB, LANE), jnp.float32),
        mesh=mesh,
        scratch_types=[
            pltpu.VMEM((SEQ_T, SUB, LANE), jnp.int32),
            pltpu.VMEM((NBUF, LANE, EMB_DIM), jnp.float32),
            pltpu.VMEM((NBUF, SUB, SUB, LANE), jnp.float32),
            pltpu.SemaphoreType.DMA((NBUF,)),
            pltpu.SemaphoreType.DMA((NBUF,)),
        ],
        compiler_params=pltpu.CompilerParams(
            use_tc_tiling_on_sc=False, needs_layout_passes=False
        ),
    )
    return run(idx4, table_p)


def kernel(inp, emb_weight):
    idx4 = inp.T.reshape(SEQ_T, SUB, BATCH_T, LANE).transpose(0, 2, 1, 3)
    table_p = (
        emb_weight.reshape(VOCAB // 16, 2, SUB, EMB_DIM)
        .transpose(0, 2, 1, 3)
        .reshape(VOCAB, EMB_DIM)
    )
    out5 = _embed(idx4, table_p)
    return out5.transpose(2, 4, 0, 1, 3).reshape(BATCH, SEQ, EMB_DIM)


# speedup vs baseline: 1.0925x; 1.0925x over previous
import jax
import jax.numpy as jnp
from jax import lax
from jax.experimental import pallas as pl
from jax.experimental.pallas import tpu as pltpu
from jax.experimental.pallas import tpu_sc as plsc

VOCAB = 1000000
EMB_DIM = 64

NC = 2
NS = 16
NW = NC * NS

BATCH = 4096
SEQ = 200
LANE = 128
SUB = 8
SEQ_T = SEQ // SUB
BATCH_T = BATCH // LANE
NBUF = 2
NROUND = SEQ // NBUF
TPITCH = LANE + 1


def _body(idx_hbm, table_hbm, out_hbm, idx_v, pbuf, gbuf, tbuf, sem_g, sem_w):
    c = lax.axis_index("c")
    s = lax.axis_index("s")
    w = s * NC + c

    pltpu.sync_copy(idx_hbm.at[:, w], idx_v)

    lanes = lax.iota(jnp.int32, 16)

    def remap_tile(si, carry):
        for sj in range(SUB):
            for k in range(LANE // 16):
                sl = pl.ds(k * 16, 16)
                pbuf[si, sj, sl] = idx_v[si, sj, sl] >> 1
        return carry

    lax.fori_loop(0, SEQ_T, remap_tile, 0)

    def gather(sq, b):
        si = sq // SUB
        sj = sq % SUB
        return pltpu.make_async_copy(
            table_hbm.at[pbuf.at[si, sj]], gbuf.at[b], sem_g.at[b]
        )

    def writeback(sq, b):
        return pltpu.make_async_copy(
            tbuf.at[b, :, :, pl.ds(0, LANE)], out_hbm.at[sq, :, w], sem_w.at[b]
        )

    def transpose(sq, b):
        si = sq // SUB
        sj = sq % SUB

        def tr_row(j, carry):
            grp = (j >> 4) << 4
            idxg = idx_v[si, sj, pl.ds(grp, 16)]
            jv = jnp.broadcast_to(j, (16,))
            hv = (idxg.at[jv & 15].get(mode="promise_in_bounds") & 1) << 6
            for m in range(EMB_DIM // 16):
                vals = plsc.load_gather(gbuf.at[b], [jv, hv + (m * 16 + lanes)])
                d = m * 16 + lanes
                plsc.store_scatter(tbuf.at[b], [d >> 3, d & 7, jv], vals)
            return carry

        lax.fori_loop(0, LANE, tr_row, 0)

    for b in range(NBUF):
        gather(b, b).start()

    def ring(i, carry):
        for b in range(NBUF):
            sq = i * NBUF + b
            gather(sq, b).wait()

            @pl.when(i > 0)
            def _():
                writeback(sq, b).wait()

            transpose(sq, b)

            @pl.when(i < NROUND - 1)
            def _():
                gather(sq + NBUF, b).start()

            writeback(sq, b).start()
        return carry

    lax.fori_loop(0, NROUND, ring, 0)

    for b in range(NBUF):
        writeback(0, b).wait()


@jax.jit
def _embed(idx4, table_f):
    mesh = plsc.VectorSubcoreMesh(core_axis_name="c", subcore_axis_name="s")
    run = pl.kernel(
        _body,
        out_type=jax.ShapeDtypeStruct((SEQ, SUB, BATCH_T, SUB, LANE), jnp.float32),
        mesh=mesh,
        scratch_types=[
            pltpu.VMEM((SEQ_T, SUB, LANE), jnp.int32),
            pltpu.VMEM((SEQ_T, SUB, LANE), jnp.int32),
            pltpu.VMEM((NBUF, LANE, LANE), jnp.float32),
            pltpu.VMEM((NBUF, SUB, SUB, TPITCH), jnp.float32),
            pltpu.SemaphoreType.DMA((NBUF,)),
            pltpu.SemaphoreType.DMA((NBUF,)),
        ],
        compiler_params=pltpu.CompilerParams(
            use_tc_tiling_on_sc=True, needs_layout_passes=False
        ),
    )
    return run(idx4, table_f)


def kernel(inp, emb_weight):
    idx4 = inp.T.reshape(SEQ_T, SUB, BATCH_T, LANE).transpose(0, 2, 1, 3)
    table_f = emb_weight.reshape(VOCAB // 2, 2 * EMB_DIM)
    out5 = _embed(idx4, table_f)
    return out5.transpose(2, 4, 0, 1, 3).reshape(BATCH, SEQ, EMB_DIM)
